# baseline (speedup 1.0000x reference)
import jax
import jax.numpy as jnp
from jax import lax
from jax.experimental import pallas as pl
from jax.experimental.pallas import tpu as pltpu
from jax.experimental.pallas import tpu_sc as plsc

_SEQ = 4096
_B = 4
_D = 1024
_NC = 2
_NS = 16
_NW = _NC * _NS

_S_CH = 4
_SEQ_W = _SEQ // _NW
_N_CH = _SEQ_W // _S_CH
_NBUF = 6


def _compute_chunk(xr, pr):
    @plsc.parallel_loop(0, _S_CH * (_D // 16), unroll=4)
    def body(n):
        s = n >> 6
        off = (n & 63) * 16
        pv = pr[s, pl.ds(off, 16)]
        for b in range(_B):
            xr[s, b, pl.ds(off, 16)] = xr[s, b, pl.ds(off, 16)] + pv


def _sc_body(x_hbm, pe_hbm, out_hbm, *scratch):
    xbufs = scratch[0:_NBUF]
    pbufs = scratch[_NBUF:2 * _NBUF]
    xsems = scratch[2 * _NBUF:3 * _NBUF]
    psems = scratch[3 * _NBUF:4 * _NBUF]
    osems = scratch[4 * _NBUF:5 * _NBUF]

    wid = lax.axis_index("s") * _NC + lax.axis_index("c")
    sbase = wid * _SEQ_W

    def load(i):
        j = i % _NBUF
        s0 = sbase + i * _S_CH
        xl = pltpu.async_copy(x_hbm.at[pl.ds(s0, _S_CH)], xbufs[j], xsems[j])
        plc = pltpu.async_copy(pe_hbm.at[pl.ds(s0, _S_CH)], pbufs[j], psems[j])
        return xl, plc

    loads = {}
    stores = {}
    loads[0] = load(0)
    for i in range(_N_CH):
        j = i % _NBUF
        if i + 1 < _N_CH:
            if (i + 1) >= _NBUF:
                stores[i + 1 - _NBUF].wait()
            loads[i + 1] = load(i + 1)
        xl, plc = loads.pop(i)
        xl.wait()
        plc.wait()
        _compute_chunk(xbufs[j], pbufs[j])
        stores[i] = pltpu.async_copy(
            xbufs[j], out_hbm.at[pl.ds(sbase + i * _S_CH, _S_CH)], osems[j])
    for i in range(_N_CH - _NBUF, _N_CH):
        if i >= 0:
            stores[i].wait()


def kernel(x, pe):
    seq_len, batch, d_model = x.shape
    mesh = plsc.VectorSubcoreMesh(core_axis_name="c", subcore_axis_name="s")
    scratch = (
        [pltpu.VMEM((_S_CH, _B, _D), jnp.float32) for _ in range(_NBUF)]
        + [pltpu.VMEM((_S_CH, _D), jnp.float32) for _ in range(_NBUF)]
        + [pltpu.SemaphoreType.DMA for _ in range(3 * _NBUF)]
    )
    f = pl.kernel(
        _sc_body,
        out_type=jax.ShapeDtypeStruct((seq_len, batch, d_model), x.dtype),
        mesh=mesh,
        scratch_types=scratch,
        compiler_params=pltpu.CompilerParams(use_tc_tiling_on_sc=True),
    )
    return f(x, pe)


# speedup vs baseline: 1.0224x; 1.0224x over previous
import jax
import jax.numpy as jnp
from jax import lax
from jax.experimental import pallas as pl
from jax.experimental.pallas import tpu as pltpu
from jax.experimental.pallas import tpu_sc as plsc

_SEQ = 4096
_B = 4
_D = 1024
_NC = 2
_NS = 16
_NW = _NC * _NS

_S_CH = 8
_SEQ_W = _SEQ // _NW
_N_CH = _SEQ_W // _S_CH
_NBUF = 3


def _compute_chunk(xr, pr):
    @plsc.parallel_loop(0, _S_CH * (_D // 16), unroll=4)
    def body(n):
        s = n >> 6
        off = (n & 63) * 16
        pv = pr[s, pl.ds(off, 16)]
        for b in range(_B):
            xr[s, b, pl.ds(off, 16)] = xr[s, b, pl.ds(off, 16)] + pv


def _sc_body(x_hbm, pe_hbm, out_hbm, *scratch):
    xbufs = scratch[0:_NBUF]
    pbufs = scratch[_NBUF:2 * _NBUF]
    xsems = scratch[2 * _NBUF:3 * _NBUF]
    psems = scratch[3 * _NBUF:4 * _NBUF]
    osems = scratch[4 * _NBUF:5 * _NBUF]

    wid = lax.axis_index("s") * _NC + lax.axis_index("c")
    sbase = wid * _SEQ_W

    def load(i):
        j = i % _NBUF
        s0 = sbase + i * _S_CH
        xl = pltpu.async_copy(x_hbm.at[pl.ds(s0, _S_CH)], xbufs[j], xsems[j])
        plc = pltpu.async_copy(pe_hbm.at[pl.ds(s0, _S_CH)], pbufs[j], psems[j])
        return xl, plc

    loads = {}
    stores = {}
    loads[0] = load(0)
    for i in range(_N_CH):
        j = i % _NBUF
        if i + 1 < _N_CH:
            if (i + 1) >= _NBUF:
                stores[i + 1 - _NBUF].wait()
            loads[i + 1] = load(i + 1)
        xl, plc = loads.pop(i)
        xl.wait()
        plc.wait()
        _compute_chunk(xbufs[j], pbufs[j])
        stores[i] = pltpu.async_copy(
            xbufs[j], out_hbm.at[pl.ds(sbase + i * _S_CH, _S_CH)], osems[j])
    for i in range(_N_CH - _NBUF, _N_CH):
        if i >= 0:
            stores[i].wait()


def kernel(x, pe):
    seq_len, batch, d_model = x.shape
    mesh = plsc.VectorSubcoreMesh(core_axis_name="c", subcore_axis_name="s")
    scratch = (
        [pltpu.VMEM((_S_CH, _B, _D), jnp.float32) for _ in range(_NBUF)]
        + [pltpu.VMEM((_S_CH, _D), jnp.float32) for _ in range(_NBUF)]
        + [pltpu.SemaphoreType.DMA for _ in range(3 * _NBUF)]
    )
    f = pl.kernel(
        _sc_body,
        out_type=jax.ShapeDtypeStruct((seq_len, batch, d_model), x.dtype),
        mesh=mesh,
        scratch_types=scratch,
        compiler_params=pltpu.CompilerParams(use_tc_tiling_on_sc=True),
    )
    return f(x, pe)


# speedup vs baseline: 1.0372x; 1.0144x over previous
import jax
import jax.numpy as jnp
from jax import lax
from jax.experimental import pallas as pl
from jax.experimental.pallas import tpu as pltpu
from jax.experimental.pallas import tpu_sc as plsc

_SEQ = 4096
_B = 4
_D = 1024
_NC = 2
_NS = 16
_NW = _NC * _NS

_S_CH = 8
_SEQ_W = _SEQ // _NW
_N_CH = _SEQ_W // _S_CH
_NBUF = 3


def _compute_chunk(xr, pr):
    @plsc.parallel_loop(0, _S_CH * (_D // 16), unroll=4)
    def body(n):
        s = n >> 6
        off = (n & 63) * 16
        pv = pr[s, pl.ds(off, 16)]
        for b in range(_B):
            xr[s, b, pl.ds(off, 16)] = xr[s, b, pl.ds(off, 16)] + pv


def _sc_body(x_hbm, pe_hbm, out_hbm, *scratch):
    xbufs = scratch[0:_NBUF]
    pbufs = scratch[_NBUF:2 * _NBUF]
    xsems = scratch[2 * _NBUF:3 * _NBUF]
    psems = scratch[3 * _NBUF:4 * _NBUF]
    osems = scratch[4 * _NBUF:5 * _NBUF]

    wid = lax.axis_index("s") * _NC + lax.axis_index("c")
    sbase = wid * _SEQ_W

    def load(i):
        j = i % _NBUF
        s0 = sbase + i * _S_CH
        xl = pltpu.async_copy(x_hbm.at[pl.ds(s0, _S_CH)], xbufs[j], xsems[j])
        plc = pltpu.async_copy(pe_hbm.at[pl.ds(s0, _S_CH)], pbufs[j], psems[j])
        return xl, plc

    loads = {}
    stores = {}
    loads[0] = load(0)
    loads[1] = load(1)
    for i in range(_N_CH):
        j = i % _NBUF
        if i + 2 < _N_CH:
            if (i + 2) >= _NBUF:
                stores[i + 2 - _NBUF].wait()
            loads[i + 2] = load(i + 2)
        xl, plc = loads.pop(i)
        xl.wait()
        plc.wait()
        _compute_chunk(xbufs[j], pbufs[j])
        stores[i] = pltpu.async_copy(
            xbufs[j], out_hbm.at[pl.ds(sbase + i * _S_CH, _S_CH)], osems[j])
    for i in range(_N_CH - _NBUF, _N_CH):
        if i >= 0:
            stores[i].wait()


def kernel(x, pe):
    seq_len, batch, d_model = x.shape
    mesh = plsc.VectorSubcoreMesh(core_axis_name="c", subcore_axis_name="s")
    scratch = (
        [pltpu.VMEM((_S_CH, _B, _D), jnp.float32) for _ in range(_NBUF)]
        + [pltpu.VMEM((_S_CH, _D), jnp.float32) for _ in range(_NBUF)]
        + [pltpu.SemaphoreType.DMA for _ in range(3 * _NBUF)]
    )
    f = pl.kernel(
        _sc_body,
        out_type=jax.ShapeDtypeStruct((seq_len, batch, d_model), x.dtype),
        mesh=mesh,
        scratch_types=scratch,
        compiler_params=pltpu.CompilerParams(use_tc_tiling_on_sc=True),
    )
    return f(x, pe)
